# baseline (speedup 1.0000x reference)
import functools

import jax
import jax.numpy as jnp
from jax import lax
from jax.experimental import pallas as pl
from jax.experimental.pallas import tpu as pltpu
from jax.experimental.pallas import tpu_sc as plsc

B, NC, NP, NE = 4, 2048, 4096, 4096
D_EMB, D_BB = 64, 512
N = B * NP
E = B * NE
NCHUNK = 8
CD = D_BB // NCHUNK
ROWS = 1024
NT = N // ROWS

@functools.cache
def _make_sc_gather_cmds():
    mesh = plsc.VectorSubcoreMesh(core_axis_name="c", subcore_axis_name="s")

    @functools.partial(
        pl.kernel,
        out_type=jax.ShapeDtypeStruct((3 * N, 16), jnp.int32),
        mesh=mesh,
        compiler_params=pltpu.CompilerParams(use_tc_tiling_on_sc=False),
        scratch_types=[
            pltpu.VMEM((16, 96), jnp.int32),
            pltpu.VMEM((1536, 16), jnp.int32),
            pltpu.SemaphoreType.DMA,
        ],
    )
    def _sc_gather_cmds_k(pidx_hbm, cmd_hbm, out_hbm, idx_v, rows_v, sem):
        w = lax.axis_index("s") * 2 + lax.axis_index("c")
        pltpu.sync_copy(pidx_hbm.at[pl.ds(w * 16, 16)], idx_v)
        handles = []
        for r in range(16):
            handles.append(
                pltpu.async_copy(cmd_hbm.at[idx_v.at[r]],
                                 rows_v.at[pl.ds(r * 96, 96)], sem))
        for h in handles:
            h.wait()
        pltpu.sync_copy(rows_v, out_hbm.at[pl.ds(w * 1536, 1536)])

    return _sc_gather_cmds_k


def _sc_gather_cmds(pidx2d, cmd_p):
    return _make_sc_gather_cmds()(pidx2d, cmd_p)


@functools.cache
def _make_sc_segment_sum():
    mesh = plsc.VectorSubcoreMesh(core_axis_name="c", subcore_axis_name="s")

    @functools.partial(
        pl.kernel,
        out_type=[
            jax.ShapeDtypeStruct((NCHUNK * N, CD), jnp.float32),
            jax.ShapeDtypeStruct((N, 16), jnp.float32),
        ],
        mesh=mesh,
        compiler_params=pltpu.CompilerParams(use_tc_tiling_on_sc=False),
        scratch_types=[
            pltpu.VMEM((8, 128), jnp.int32),
            pltpu.VMEM((8, 128), jnp.int32),
            pltpu.VMEM((8, 128), jnp.int32),
            pltpu.VMEM((128, CD), jnp.float32),
            pltpu.VMEM((512, CD), jnp.float32),
            pltpu.VMEM((256, 16), jnp.float32),
            pltpu.VMEM_SHARED((N, CD), jnp.float32),
            pltpu.VMEM_SHARED((N, 16), jnp.float32),
            pltpu.SemaphoreType.DMA,
        ],
    )
    def _sc_segment_sum_k(src_hbm, dst_hbm, m_hbm, out_hbm, cnt_hbm,
                          src_v, dst_v, gidx_v, zero_v, rows_v, const_v,
                          accum, cnt_acc, sem):
        c = lax.axis_index("c")
        t = lax.axis_index("s")
        pltpu.sync_copy(src_hbm.at[pl.ds(t * 8, 8)], src_v)
        pltpu.sync_copy(dst_hbm.at[pl.ds(t * 8, 8)], dst_v)

        zv = jnp.zeros((16,), jnp.float32)
        ov = jnp.ones((16,), jnp.float32)

        def _zero_row(i, carry):
            for k in range(CD // 16):
                zero_v[i, pl.ds(k * 16, 16)] = zv
            const_v[i, :] = ov
            const_v[i + 128, :] = zv
            return carry

        lax.fori_loop(0, 128, _zero_row, 0)

        for cc in range(4):
            chunk_off = (c * 4 + cc) * N
            for g in range(8):
                for k in range(8):
                    gidx_v[g, pl.ds(k * 16, 16)] = (
                        src_v[g, pl.ds(k * 16, 16)] + chunk_off)
            handles = [
                pltpu.async_copy(m_hbm.at[gidx_v.at[g]],
                                 rows_v.at[pl.ds(g * 128, 128)], sem)
                for g in range(4)]
            for z in range(8):
                pltpu.sync_copy(zero_v,
                                accum.at[pl.ds(t * 1024 + z * 128, 128)])
            if cc == 0:
                for z in range(8):
                    pltpu.sync_copy(
                        const_v.at[pl.ds(128, 128)],
                        cnt_acc.at[pl.ds(t * 1024 + z * 128, 128)])
            plsc.subcore_barrier()
            for g in range(8):
                slot = g % 4
                handles[slot].wait()
                pltpu.sync_copy(rows_v.at[pl.ds(slot * 128, 128)],
                                accum.at[dst_v.at[g]], add=True)
                if g + 4 < 8:
                    handles[slot] = pltpu.async_copy(
                        m_hbm.at[gidx_v.at[g + 4]],
                        rows_v.at[pl.ds(slot * 128, 128)], sem)
            if cc == 0:
                for g in range(8):
                    pltpu.sync_copy(const_v.at[pl.ds(0, 128)],
                                    cnt_acc.at[dst_v.at[g]], add=True)
            plsc.subcore_barrier()
            pltpu.sync_copy(accum.at[pl.ds(t * 1024, 1024)],
                            out_hbm.at[pl.ds(chunk_off + t * 1024, 1024)])
            if cc == 0:

                @pl.when(c == 0)
                def _():
                    pltpu.sync_copy(cnt_acc.at[pl.ds(t * 1024, 1024)],
                                    cnt_hbm.at[pl.ds(t * 1024, 1024)])

    return _sc_segment_sum_k


def _sc_segment_sum(src2d, dst2d, m_flat):
    return _make_sc_segment_sum()(src2d, dst2d, m_flat)


def _k1_body(pcp_ref, a_ref, w3_ref, bin_ref, wp_ref, bp_ref, wr_ref,
             blr_ref, m8_ref, z_ref, u1_s, u2_s, b0_s):
    @pl.when(pl.program_id(0) == 0)
    def _prologue():
        u1_s[...] = jnp.zeros((48, D_BB), jnp.bfloat16)
        u2_s[...] = jnp.zeros((48, D_BB), jnp.bfloat16)
        b0 = bin_ref[...]
        for j in range(27):
            t8 = jnp.dot(a_ref[j], w3_ref[j],
                         preferred_element_type=jnp.float32)
            row = (j // 9) * 16 + (j % 9)
            t0 = t8[0:1]
            u1_s[row:row + 1, :] = (t8[1:2] - t0).astype(jnp.bfloat16)
            u2_s[row:row + 1, :] = (t8[2:3] - t0).astype(jnp.bfloat16)
            b0 = b0 + t0
        b0_s[...] = b0

    idx = pcp_ref[...]
    c1 = (idx == 1).astype(jnp.bfloat16)
    c2 = (idx == 2).astype(jnp.bfloat16)
    x = (jnp.dot(c1, u1_s[...], preferred_element_type=jnp.float32)
         + jnp.dot(c2, u2_s[...], preferred_element_type=jnp.float32)
         + b0_s[...])
    xb = x.astype(jnp.bfloat16)
    z_ref[...] = (jnp.dot(xb, wr_ref[...],
                          preferred_element_type=jnp.float32)
                  + blr_ref[...]).astype(jnp.bfloat16)
    m = jnp.maximum(
        jnp.dot(xb, wp_ref[...], preferred_element_type=jnp.float32)
        + bp_ref[...], 0.0)
    for cch in range(NCHUNK):
        m8_ref[cch] = m[:, cch * CD:(cch + 1) * CD]


def _k1(pcp48, A_pad, W3, b_in, Wp, bp, Wr, blr):
    return pl.pallas_call(
        _k1_body,
        grid=(NT,),
        in_specs=[
            pl.BlockSpec((ROWS, 48), lambda i: (i, 0)),
            pl.BlockSpec((27, 8, D_EMB), lambda i: (0, 0, 0)),
            pl.BlockSpec((27, D_EMB, D_BB), lambda i: (0, 0, 0)),
            pl.BlockSpec((1, D_BB), lambda i: (0, 0)),
            pl.BlockSpec((D_BB, D_BB), lambda i: (0, 0)),
            pl.BlockSpec((1, D_BB), lambda i: (0, 0)),
            pl.BlockSpec((D_BB, D_BB), lambda i: (0, 0)),
            pl.BlockSpec((1, D_BB), lambda i: (0, 0)),
        ],
        out_specs=[
            pl.BlockSpec((NCHUNK, ROWS, CD), lambda i: (0, i, 0)),
            pl.BlockSpec((ROWS, D_BB), lambda i: (i, 0)),
        ],
        out_shape=[
            jax.ShapeDtypeStruct((NCHUNK, N, CD), jnp.float32),
            jax.ShapeDtypeStruct((N, D_BB), jnp.bfloat16),
        ],
        scratch_shapes=[
            pltpu.VMEM((48, D_BB), jnp.bfloat16),
            pltpu.VMEM((48, D_BB), jnp.bfloat16),
            pltpu.VMEM((1, D_BB), jnp.float32),
        ],
    )(pcp48, A_pad, W3, b_in, Wp, bp, Wr, blr)


def _sage_tail(s8, inv, z, wl_ref):
    out = z.astype(jnp.float32)
    for cch in range(NCHUNK):
        aggc = (s8[cch] * inv).astype(jnp.bfloat16)
        out = out + jnp.dot(aggc, wl_ref[pl.ds(cch * CD, CD), :],
                            preferred_element_type=jnp.float32)
    nrm = jnp.sqrt(jnp.sum(out * out, axis=-1, keepdims=True))
    return out / jnp.maximum(nrm, 1e-12)


def _k3_body(s8_ref, cnt_ref, z_ref, wl_ref, wp_ref, bp_ref, wr_ref,
             blr_ref, m8_ref, z2_ref):
    inv = 1.0 / jnp.maximum(cnt_ref[...][:, 0:1], 1.0)
    h = _sage_tail(s8_ref[...], inv, z_ref[...], wl_ref)
    hb = h.astype(jnp.bfloat16)
    z2_ref[...] = (jnp.dot(hb, wr_ref[...],
                           preferred_element_type=jnp.float32)
                   + blr_ref[...]).astype(jnp.bfloat16)
    m = jnp.maximum(
        jnp.dot(hb, wp_ref[...], preferred_element_type=jnp.float32)
        + bp_ref[...], 0.0)
    for cch in range(NCHUNK):
        m8_ref[cch] = m[:, cch * CD:(cch + 1) * CD]


def _k3(s8, cnt16, z, Wl, Wp, bp, Wr, blr):
    return pl.pallas_call(
        _k3_body,
        grid=(NT,),
        in_specs=[
            pl.BlockSpec((NCHUNK, ROWS, CD), lambda i: (0, i, 0)),
            pl.BlockSpec((ROWS, 16), lambda i: (i, 0)),
            pl.BlockSpec((ROWS, D_BB), lambda i: (i, 0)),
            pl.BlockSpec((D_BB, D_BB), lambda i: (0, 0)),
            pl.BlockSpec((D_BB, D_BB), lambda i: (0, 0)),
            pl.BlockSpec((1, D_BB), lambda i: (0, 0)),
            pl.BlockSpec((D_BB, D_BB), lambda i: (0, 0)),
            pl.BlockSpec((1, D_BB), lambda i: (0, 0)),
        ],
        out_specs=[
            pl.BlockSpec((NCHUNK, ROWS, CD), lambda i: (0, i, 0)),
            pl.BlockSpec((ROWS, D_BB), lambda i: (i, 0)),
        ],
        out_shape=[
            jax.ShapeDtypeStruct((NCHUNK, N, CD), jnp.float32),
            jax.ShapeDtypeStruct((N, D_BB), jnp.bfloat16),
        ],
    )(s8, cnt16, z, Wl, Wp, bp, Wr, blr)


def _k5_body(s8_ref, cnt_ref, z_ref, wl_ref, g_ref, be_ref,
             y_ref):
    inv = 1.0 / jnp.maximum(cnt_ref[...][:, 0:1], 1.0)
    h2 = _sage_tail(s8_ref[...], inv, z_ref[...], wl_ref)
    mu = jnp.mean(h2, axis=-1, keepdims=True)
    d = h2 - mu
    var = jnp.mean(d * d, axis=-1, keepdims=True)
    y_ref[...] = d * jax.lax.rsqrt(var + 1e-5) * g_ref[...] + be_ref[...]


def _k5(s8, cnt16, z, Wl, gamma, beta):
    return pl.pallas_call(
        _k5_body,
        grid=(NT,),
        in_specs=[
            pl.BlockSpec((NCHUNK, ROWS, CD), lambda i: (0, i, 0)),
            pl.BlockSpec((ROWS, 16), lambda i: (i, 0)),
            pl.BlockSpec((ROWS, D_BB), lambda i: (i, 0)),
            pl.BlockSpec((D_BB, D_BB), lambda i: (0, 0)),
            pl.BlockSpec((1, D_BB), lambda i: (0, 0)),
            pl.BlockSpec((1, D_BB), lambda i: (0, 0)),
        ],
        out_specs=pl.BlockSpec((ROWS, D_BB), lambda i: (i, 0)),
        out_shape=jax.ShapeDtypeStruct((N, D_BB), jnp.float32),
    )(s8, cnt16, z, Wl, gamma, beta)


def kernel(svg_commands, svg_paths, svg_edges, type_table, coor_table,
           W_in, b_in, W_proj0, b_proj0, W_l0, b_l0, W_r0,
           W_proj1, b_proj1, W_l1, b_l1, W_r1, gamma, beta):
    cmd_p = jnp.pad(svg_commands.reshape(B * NC, 9), ((0, 0), (0, 7)))
    pidx = (svg_paths.reshape(B, NP * 3)
            + (jnp.arange(B, dtype=jnp.int32) * NC)[:, None])
    pidx2d = pidx.reshape(512, 96)
    offs = (jnp.arange(B, dtype=jnp.int32) * NP)[:, None]
    src2d = (svg_edges[..., 0] + offs).reshape(128, 128)
    dst2d = (svg_edges[..., 1] + offs).reshape(128, 128)

    A = jnp.stack([type_table if j % 9 == 0 else coor_table[:3]
                   for j in range(27)])
    A_pad = jnp.pad(A, ((0, 0), (0, 5), (0, 0)))
    W3 = W_in.reshape(27, D_EMB, D_BB)

    pcp = _sc_gather_cmds(pidx2d, cmd_p)
    pcp48 = pcp.reshape(N, 48)

    bf = jnp.bfloat16
    m8_0, z0 = _k1(pcp48, A_pad, W3, b_in[None, :], W_proj0.astype(bf),
                   b_proj0[None, :], W_r0.astype(bf), b_l0[None, :])

    s0, cnt16 = _sc_segment_sum(src2d, dst2d, m8_0.reshape(NCHUNK * N, CD))
    m8_1, z1 = _k3(s0.reshape(NCHUNK, N, CD), cnt16, z0,
                   W_l0.astype(bf), W_proj1.astype(bf), b_proj1[None, :],
                   W_r1.astype(bf), b_l1[None, :])

    s1, _ = _sc_segment_sum(src2d, dst2d, m8_1.reshape(NCHUNK * N, CD))
    y = _k5(s1.reshape(NCHUNK, N, CD), cnt16, z1,
            W_l1.astype(bf), gamma[None, :], beta[None, :])
    return y.reshape(B, NP, D_BB)


# speedup vs baseline: 1.0192x; 1.0192x over previous
import functools

import jax
import jax.numpy as jnp
from jax import lax
from jax.experimental import pallas as pl
from jax.experimental.pallas import tpu as pltpu
from jax.experimental.pallas import tpu_sc as plsc

B, NC, NP, NE = 4, 2048, 4096, 4096
D_EMB, D_BB = 64, 512
N = B * NP
E = B * NE
NCHUNK = 8
CD = D_BB // NCHUNK
ROWS = 1024
NT = N // ROWS

@functools.cache
def _make_sc_gather_cmds():
    mesh = plsc.VectorSubcoreMesh(core_axis_name="c", subcore_axis_name="s")

    @functools.partial(
        pl.kernel,
        out_type=jax.ShapeDtypeStruct((3 * N, 16), jnp.int32),
        mesh=mesh,
        compiler_params=pltpu.CompilerParams(use_tc_tiling_on_sc=False),
        scratch_types=[
            pltpu.VMEM((16, 96), jnp.int32),
            pltpu.VMEM((1536, 16), jnp.int32),
            pltpu.SemaphoreType.DMA,
        ],
    )
    def _sc_gather_cmds_k(pidx_hbm, cmd_hbm, out_hbm, idx_v, rows_v, sem):
        w = lax.axis_index("s") * 2 + lax.axis_index("c")
        pltpu.sync_copy(pidx_hbm.at[pl.ds(w * 16, 16)], idx_v)
        handles = []
        for r in range(16):
            handles.append(
                pltpu.async_copy(cmd_hbm.at[idx_v.at[r]],
                                 rows_v.at[pl.ds(r * 96, 96)], sem))
        for h in handles:
            h.wait()
        pltpu.sync_copy(rows_v, out_hbm.at[pl.ds(w * 1536, 1536)])

    return _sc_gather_cmds_k


def _sc_gather_cmds(pidx2d, cmd_p):
    return _make_sc_gather_cmds()(pidx2d, cmd_p)


@functools.cache
def _make_sc_segment_sum():
    mesh = plsc.VectorSubcoreMesh(core_axis_name="c", subcore_axis_name="s")

    @functools.partial(
        pl.kernel,
        out_type=[
            jax.ShapeDtypeStruct((NCHUNK * N, CD), jnp.float32),
            jax.ShapeDtypeStruct((N, 16), jnp.float32),
        ],
        mesh=mesh,
        compiler_params=pltpu.CompilerParams(use_tc_tiling_on_sc=False),
        scratch_types=[
            pltpu.VMEM((8, 128), jnp.int32),
            pltpu.VMEM((8, 128), jnp.int32),
            pltpu.VMEM((8, 128), jnp.int32),
            pltpu.VMEM((128, CD), jnp.float32),
            pltpu.VMEM((512, CD), jnp.float32),
            pltpu.VMEM((256, 16), jnp.float32),
            pltpu.VMEM_SHARED((N, CD), jnp.float32),
            pltpu.VMEM_SHARED((N, 16), jnp.float32),
            pltpu.SemaphoreType.DMA,
        ],
    )
    def _sc_segment_sum_k(src_hbm, dst_hbm, m_hbm, out_hbm, cnt_hbm,
                          src_v, dst_v, gidx_v, zero_v, rows_v, const_v,
                          accum, cnt_acc, sem):
        c = lax.axis_index("c")
        t = lax.axis_index("s")
        pltpu.sync_copy(src_hbm.at[pl.ds(t * 8, 8)], src_v)
        pltpu.sync_copy(dst_hbm.at[pl.ds(t * 8, 8)], dst_v)

        zv = jnp.zeros((16,), jnp.float32)
        ov = jnp.ones((16,), jnp.float32)

        def _zero_row(i, carry):
            for k in range(CD // 16):
                zero_v[i, pl.ds(k * 16, 16)] = zv
            const_v[i, :] = ov
            const_v[i + 128, :] = zv
            return carry

        lax.fori_loop(0, 128, _zero_row, 0)

        for cc in range(4):
            chunk_off = (c * 4 + cc) * N
            for g in range(8):
                for k in range(8):
                    gidx_v[g, pl.ds(k * 16, 16)] = (
                        src_v[g, pl.ds(k * 16, 16)] + chunk_off)
            handles = [
                pltpu.async_copy(m_hbm.at[gidx_v.at[g]],
                                 rows_v.at[pl.ds(g * 128, 128)], sem)
                for g in range(4)]
            for z in range(8):
                pltpu.sync_copy(zero_v,
                                accum.at[pl.ds(t * 1024 + z * 128, 128)])
            if cc == 0:
                for z in range(8):
                    pltpu.sync_copy(
                        const_v.at[pl.ds(128, 128)],
                        cnt_acc.at[pl.ds(t * 1024 + z * 128, 128)])
            plsc.subcore_barrier()
            for g in range(8):
                slot = g % 4
                handles[slot].wait()
                pltpu.sync_copy(rows_v.at[pl.ds(slot * 128, 128)],
                                accum.at[dst_v.at[g]], add=True)
                if g + 4 < 8:
                    handles[slot] = pltpu.async_copy(
                        m_hbm.at[gidx_v.at[g + 4]],
                        rows_v.at[pl.ds(slot * 128, 128)], sem)
            if cc == 0:
                for g in range(8):
                    pltpu.sync_copy(const_v.at[pl.ds(0, 128)],
                                    cnt_acc.at[dst_v.at[g]], add=True)
            plsc.subcore_barrier()
            pltpu.sync_copy(accum.at[pl.ds(t * 1024, 1024)],
                            out_hbm.at[pl.ds(chunk_off + t * 1024, 1024)])
            if cc == 0:

                @pl.when(c == 0)
                def _():
                    pltpu.sync_copy(cnt_acc.at[pl.ds(t * 1024, 1024)],
                                    cnt_hbm.at[pl.ds(t * 1024, 1024)])

    return _sc_segment_sum_k


def _sc_segment_sum(src2d, dst2d, m_flat):
    return _make_sc_segment_sum()(src2d, dst2d, m_flat)


def _k1_body(pcp_ref, a_ref, w3_ref, bin_ref, wp_ref, bp_ref,
             x_ref, m8_ref, u1_s, u2_s, b0_s):
    @pl.when(pl.program_id(0) == 0)
    def _prologue():
        u1_s[...] = jnp.zeros((48, D_BB), jnp.bfloat16)
        u2_s[...] = jnp.zeros((48, D_BB), jnp.bfloat16)
        b0 = bin_ref[...]
        for j in range(27):
            t8 = jnp.dot(a_ref[j], w3_ref[j],
                         preferred_element_type=jnp.float32)
            row = (j // 9) * 16 + (j % 9)
            t0 = t8[0:1]
            u1_s[row:row + 1, :] = (t8[1:2] - t0).astype(jnp.bfloat16)
            u2_s[row:row + 1, :] = (t8[2:3] - t0).astype(jnp.bfloat16)
            b0 = b0 + t0
        b0_s[...] = b0

    idx = pcp_ref[...]
    c1 = (idx == 1).astype(jnp.bfloat16)
    c2 = (idx == 2).astype(jnp.bfloat16)
    x = (jnp.dot(c1, u1_s[...], preferred_element_type=jnp.float32)
         + jnp.dot(c2, u2_s[...], preferred_element_type=jnp.float32)
         + b0_s[...])
    xb = x.astype(jnp.bfloat16)
    x_ref[...] = xb
    m = jnp.maximum(
        jnp.dot(xb, wp_ref[...], preferred_element_type=jnp.float32)
        + bp_ref[...], 0.0)
    for cch in range(NCHUNK):
        m8_ref[cch] = m[:, cch * CD:(cch + 1) * CD]


def _k1(pcp48, A_pad, W3, b_in, Wp, bp):
    return pl.pallas_call(
        _k1_body,
        grid=(NT,),
        in_specs=[
            pl.BlockSpec((ROWS, 48), lambda i: (i, 0)),
            pl.BlockSpec((27, 8, D_EMB), lambda i: (0, 0, 0)),
            pl.BlockSpec((27, D_EMB, D_BB), lambda i: (0, 0, 0)),
            pl.BlockSpec((1, D_BB), lambda i: (0, 0)),
            pl.BlockSpec((D_BB, D_BB), lambda i: (0, 0)),
            pl.BlockSpec((1, D_BB), lambda i: (0, 0)),
        ],
        out_specs=[
            pl.BlockSpec((ROWS, D_BB), lambda i: (i, 0)),
            pl.BlockSpec((NCHUNK, ROWS, CD), lambda i: (0, i, 0)),
        ],
        out_shape=[
            jax.ShapeDtypeStruct((N, D_BB), jnp.bfloat16),
            jax.ShapeDtypeStruct((NCHUNK, N, CD), jnp.float32),
        ],
        scratch_shapes=[
            pltpu.VMEM((48, D_BB), jnp.bfloat16),
            pltpu.VMEM((48, D_BB), jnp.bfloat16),
            pltpu.VMEM((1, D_BB), jnp.float32),
        ],
    )(pcp48, A_pad, W3, b_in, Wp, bp)


def _kz_body(h_ref, wr_ref, bl_ref, z_ref):
    z_ref[...] = (jnp.dot(h_ref[...], wr_ref[...],
                          preferred_element_type=jnp.float32)
                  + bl_ref[...]).astype(jnp.bfloat16)


def _kz(h, Wr, bl):
    return pl.pallas_call(
        _kz_body,
        grid=(NT,),
        in_specs=[
            pl.BlockSpec((ROWS, D_BB), lambda i: (i, 0)),
            pl.BlockSpec((D_BB, D_BB), lambda i: (0, 0)),
            pl.BlockSpec((1, D_BB), lambda i: (0, 0)),
        ],
        out_specs=pl.BlockSpec((ROWS, D_BB), lambda i: (i, 0)),
        out_shape=jax.ShapeDtypeStruct((N, D_BB), jnp.bfloat16),
    )(h, Wr, bl)


def _sage_tail(s8, inv, z, wl_ref):
    out = z.astype(jnp.float32)
    for cch in range(NCHUNK):
        aggc = (s8[cch] * inv).astype(jnp.bfloat16)
        out = out + jnp.dot(aggc, wl_ref[pl.ds(cch * CD, CD), :],
                            preferred_element_type=jnp.float32)
    nrm = jnp.sqrt(jnp.sum(out * out, axis=-1, keepdims=True))
    return out / jnp.maximum(nrm, 1e-12)


def _k3_body(s8_ref, cnt_ref, z_ref, wl_ref, wp_ref, bp_ref,
             h_ref, m8_ref):
    inv = 1.0 / jnp.maximum(cnt_ref[...][:, 0:1], 1.0)
    h = _sage_tail(s8_ref[...], inv, z_ref[...], wl_ref)
    hb = h.astype(jnp.bfloat16)
    h_ref[...] = hb
    m = jnp.maximum(
        jnp.dot(hb, wp_ref[...], preferred_element_type=jnp.float32)
        + bp_ref[...], 0.0)
    for cch in range(NCHUNK):
        m8_ref[cch] = m[:, cch * CD:(cch + 1) * CD]


def _k3(s8, cnt16, z, Wl, Wp, bp):
    return pl.pallas_call(
        _k3_body,
        grid=(NT,),
        in_specs=[
            pl.BlockSpec((NCHUNK, ROWS, CD), lambda i: (0, i, 0)),
            pl.BlockSpec((ROWS, 16), lambda i: (i, 0)),
            pl.BlockSpec((ROWS, D_BB), lambda i: (i, 0)),
            pl.BlockSpec((D_BB, D_BB), lambda i: (0, 0)),
            pl.BlockSpec((D_BB, D_BB), lambda i: (0, 0)),
            pl.BlockSpec((1, D_BB), lambda i: (0, 0)),
        ],
        out_specs=[
            pl.BlockSpec((ROWS, D_BB), lambda i: (i, 0)),
            pl.BlockSpec((NCHUNK, ROWS, CD), lambda i: (0, i, 0)),
        ],
        out_shape=[
            jax.ShapeDtypeStruct((N, D_BB), jnp.bfloat16),
            jax.ShapeDtypeStruct((NCHUNK, N, CD), jnp.float32),
        ],
    )(s8, cnt16, z, Wl, Wp, bp)


def _k5_body(s8_ref, cnt_ref, z_ref, wl_ref, g_ref, be_ref,
             y_ref):
    inv = 1.0 / jnp.maximum(cnt_ref[...][:, 0:1], 1.0)
    h2 = _sage_tail(s8_ref[...], inv, z_ref[...], wl_ref)
    mu = jnp.mean(h2, axis=-1, keepdims=True)
    d = h2 - mu
    var = jnp.mean(d * d, axis=-1, keepdims=True)
    y_ref[...] = d * jax.lax.rsqrt(var + 1e-5) * g_ref[...] + be_ref[...]


def _k5(s8, cnt16, z, Wl, gamma, beta):
    return pl.pallas_call(
        _k5_body,
        grid=(NT,),
        in_specs=[
            pl.BlockSpec((NCHUNK, ROWS, CD), lambda i: (0, i, 0)),
            pl.BlockSpec((ROWS, 16), lambda i: (i, 0)),
            pl.BlockSpec((ROWS, D_BB), lambda i: (i, 0)),
            pl.BlockSpec((D_BB, D_BB), lambda i: (0, 0)),
            pl.BlockSpec((1, D_BB), lambda i: (0, 0)),
            pl.BlockSpec((1, D_BB), lambda i: (0, 0)),
        ],
        out_specs=pl.BlockSpec((ROWS, D_BB), lambda i: (i, 0)),
        out_shape=jax.ShapeDtypeStruct((N, D_BB), jnp.float32),
    )(s8, cnt16, z, Wl, gamma, beta)


def kernel(svg_commands, svg_paths, svg_edges, type_table, coor_table,
           W_in, b_in, W_proj0, b_proj0, W_l0, b_l0, W_r0,
           W_proj1, b_proj1, W_l1, b_l1, W_r1, gamma, beta):
    cmd_p = jnp.pad(svg_commands.reshape(B * NC, 9), ((0, 0), (0, 7)))
    pidx = (svg_paths.reshape(B, NP * 3)
            + (jnp.arange(B, dtype=jnp.int32) * NC)[:, None])
    pidx2d = pidx.reshape(512, 96)
    offs = (jnp.arange(B, dtype=jnp.int32) * NP)[:, None]
    src2d = (svg_edges[..., 0] + offs).reshape(128, 128)
    dst2d = (svg_edges[..., 1] + offs).reshape(128, 128)

    A = jnp.stack([type_table if j % 9 == 0 else coor_table[:3]
                   for j in range(27)])
    A_pad = jnp.pad(A, ((0, 0), (0, 5), (0, 0)))
    W3 = W_in.reshape(27, D_EMB, D_BB)

    pcp = _sc_gather_cmds(pidx2d, cmd_p)
    pcp48 = pcp.reshape(N, 48)

    bf = jnp.bfloat16
    x, m8_0 = _k1(pcp48, A_pad, W3, b_in[None, :], W_proj0.astype(bf),
                  b_proj0[None, :])

    s0, cnt16 = _sc_segment_sum(src2d, dst2d, m8_0.reshape(NCHUNK * N, CD))
    z0 = _kz(x, W_r0.astype(bf), b_l0[None, :])
    h1, m8_1 = _k3(s0.reshape(NCHUNK, N, CD), cnt16, z0,
                   W_l0.astype(bf), W_proj1.astype(bf), b_proj1[None, :])

    s1, _ = _sc_segment_sum(src2d, dst2d, m8_1.reshape(NCHUNK * N, CD))
    z1 = _kz(h1, W_r1.astype(bf), b_l1[None, :])
    y = _k5(s1.reshape(NCHUNK, N, CD), cnt16, z1,
            W_l1.astype(bf), gamma[None, :], beta[None, :])
    return y.reshape(B, NP, D_BB)
